# baseline (speedup 1.0000x reference)
import functools

import jax
import jax.numpy as jnp
from jax import lax
from jax.experimental import pallas as pl
from jax.experimental.pallas import tpu as pltpu
from jax.experimental.pallas import tpu_sc as plsc

NC = 2
NS = 16
NW = NC * NS
LANES = 16


def _make_lookup(B, T, V, D):
    total = B * T
    per_w = total // NW
    C = 2 * T
    G = 80
    KG = C // G
    NCH = per_w // C
    KV = D // LANES
    REP = C // T

    mesh = plsc.VectorSubcoreMesh(core_axis_name="c", subcore_axis_name="s")

    @functools.partial(
        pl.kernel,
        mesh=mesh,
        out_type=jax.ShapeDtypeStruct((total, 2 * D), jnp.float32),
        compiler_params=pltpu.CompilerParams(use_tc_tiling_on_sc=False),
        scratch_types=[
            pltpu.VMEM((2 * C + LANES,), jnp.int32),
            pltpu.VMEM((2 * C,), jnp.int32),
            pltpu.VMEM((2, C, 2 * D), jnp.float32),
            pltpu.VMEM((T, D), jnp.float32),
            pltpu.SemaphoreType.DMA,
            pltpu.SemaphoreType.DMA,
            pltpu.SemaphoreType.DMA,
        ],
    )
    def lookup(x_hbm, emb2_hbm, pos_hbm, out_hbm, idxr_v, idxh_v, slab_v,
               pos_v, isem, gsem, wsem):
        wid = lax.axis_index("s") * NC + lax.axis_index("c")
        base_w = wid * per_w
        pltpu.sync_copy(pos_hbm, pos_v)

        def idx_copy(c, buf):
            return pltpu.make_async_copy(
                x_hbm.at[pl.ds(base_w + c * C, C)],
                idxr_v.at[pl.ds(buf * C, C)],
                isem,
            )

        def halve_ids(buf):
            for v in range(C // LANES):
                sl = pl.ds(buf * C + v * LANES, LANES)
                idxh_v[sl] = lax.shift_right_logical(idxr_v[sl], 1)

        def gather_copies(buf):
            return [
                pltpu.make_async_copy(
                    emb2_hbm.at[idxh_v.at[pl.ds(buf * C + j * G, G)]],
                    slab_v.at[buf, pl.ds(j * G, G)],
                    gsem,
                )
                for j in range(KG)
            ]

        def write_copy(c, buf):
            return pltpu.make_async_copy(
                slab_v.at[buf],
                out_hbm.at[pl.ds(base_w + c * C, C)],
                wsem,
            )

        idx_copy(0, 0).start()
        idx_copy(0, 0).wait()
        halve_ids(0)
        for cp in gather_copies(0):
            cp.start()

        def chunk_body(c, carry):
            b0 = lax.rem(c, 2)
            b1 = lax.rem(c + 1, 2)

            @pl.when(c + 1 < NCH)
            def _():
                idx_copy(c + 1, b1).start()

            for cp in gather_copies(b0):
                cp.wait()

            @pl.when(c + 1 < NCH)
            def _():
                idx_copy(c + 1, b1).wait()
                halve_ids(b1)

                @pl.when(c > 0)
                def _():
                    write_copy(c - 1, b1).wait()

                for cp in gather_copies(b1):
                    cp.start()

            def g_body(g, carry2):
                pv = idxr_v[pl.ds(b0 * C + g * LANES, LANES)] & 1
                for i in range(LANES):
                    r = g * LANES + i

                    @pl.when(pv[i] > 0)
                    def _():
                        for k in range(KV):
                            slab_v[b0, r, pl.ds(k * LANES, LANES)] = (
                                slab_v[b0, r, pl.ds(D + k * LANES, LANES)]
                            )
                return carry2

            lax.fori_loop(0, C // LANES, g_body, 0)

            def t_body(t, carry2):
                pvec = [pos_v[t, pl.ds(k * LANES, LANES)] for k in range(KV)]
                for rep in range(REP):
                    r = rep * T + t
                    for k in range(KV):
                        sl = pl.ds(k * LANES, LANES)
                        slab_v[b0, r, sl] = slab_v[b0, r, sl] + pvec[k]
                return carry2

            lax.fori_loop(0, T, t_body, 0)
            write_copy(c, b0).start()
            return carry

        lax.fori_loop(0, NCH, chunk_body, 0)
        write_copy(NCH - 2, (NCH - 2) % 2).wait()
        write_copy(NCH - 1, (NCH - 1) % 2).wait()

    return lookup


def kernel(x, emb, posenc):
    B, T = x.shape
    V, D = emb.shape
    x1 = x.astype(jnp.int32).reshape(-1)
    emb2 = emb.reshape(V // 2, 2 * D)
    posd = posenc[:T]
    out = _make_lookup(B, T, V, D)(x1, emb2, posd)
    return out.reshape(B, T, 2 * D)[:, :, :D]


# speedup vs baseline: 1.2364x; 1.2364x over previous
import functools

import jax
import jax.numpy as jnp
from jax import lax
from jax.experimental import pallas as pl
from jax.experimental.pallas import tpu as pltpu
from jax.experimental.pallas import tpu_sc as plsc

NC = 2
NS = 16
LANES = 16


def kernel(x, emb, posenc):
    B, T = x.shape
    V, D = emb.shape
    total = B * T
    NW = NC * NS
    per_w = total // NW
    C = 4 * T
    G = 100
    KG = C // G
    NCH = per_w // C
    KV = D // LANES
    REP = C // T

    xf = x.astype(jnp.int32).reshape(total // G, G)

    mesh = plsc.VectorSubcoreMesh(core_axis_name="c", subcore_axis_name="s")

    @functools.partial(
        pl.kernel,
        mesh=mesh,
        out_type=jax.ShapeDtypeStruct((total, D), jnp.float32),
        compiler_params=pltpu.CompilerParams(use_tc_tiling_on_sc=False),
        scratch_types=[
            pltpu.VMEM((2, KG, G), jnp.int32),
            pltpu.VMEM((2, C, D), jnp.float32),
            pltpu.VMEM((T, D), jnp.float32),
            pltpu.SemaphoreType.DMA,
            pltpu.SemaphoreType.DMA,
            pltpu.SemaphoreType.DMA,
        ],
    )
    def run(x_hbm, emb_hbm, pos_hbm, out_hbm, idx_v, rows_v, pos_v,
            isem, gsem, wsem):
        wid = lax.axis_index("s") * NC + lax.axis_index("c")
        row0_w = wid * (per_w // G)
        base_w = wid * per_w
        pltpu.sync_copy(pos_hbm.at[pl.ds(0, T)], pos_v)

        def start_gathers(c, buf):
            for j in range(KG):
                pltpu.make_async_copy(
                    emb_hbm.at[idx_v.at[buf].at[j]],
                    rows_v.at[buf].at[pl.ds(j * G, G)],
                    gsem,
                ).start()

        def wait_gathers(buf):
            for j in range(KG):
                pltpu.make_async_copy(
                    emb_hbm.at[idx_v.at[buf].at[j]],
                    rows_v.at[buf].at[pl.ds(j * G, G)],
                    gsem,
                ).wait()

        def write_copy(c, buf):
            return pltpu.make_async_copy(
                rows_v.at[buf],
                out_hbm.at[pl.ds(base_w + c * C, C)],
                wsem,
            )

        pltpu.sync_copy(x_hbm.at[pl.ds(row0_w, KG)], idx_v.at[0])
        start_gathers(0, 0)

        def chunk_body(c, carry):
            b0 = lax.rem(c, 2)
            b1 = lax.rem(c + 1, 2)

            @pl.when(c + 1 < NCH)
            def _():
                pltpu.make_async_copy(
                    x_hbm.at[pl.ds(row0_w + (c + 1) * KG, KG)],
                    idx_v.at[b1],
                    isem,
                ).start()

            wait_gathers(b0)

            @pl.when(c + 1 < NCH)
            def _():
                pltpu.make_async_copy(
                    x_hbm.at[pl.ds(row0_w + (c + 1) * KG, KG)],
                    idx_v.at[b1],
                    isem,
                ).wait()

                @pl.when(c > 0)
                def _():
                    write_copy(c - 1, b1).wait()

                start_gathers(c + 1, b1)

            @plsc.parallel_loop(0, T, unroll=2)
            def t_body(t):
                pvec = [pos_v[t, pl.ds(k * LANES, LANES)] for k in range(KV)]
                for rep in range(REP):
                    r = rep * T + t
                    for k in range(KV):
                        sl = pl.ds(k * LANES, LANES)
                        rows_v[b0, r, sl] = rows_v[b0, r, sl] + pvec[k]

            write_copy(c, b0).start()
            return carry

        lax.fori_loop(0, NCH, chunk_body, 0)
        write_copy(NCH - 2, (NCH - 2) % 2).wait()
        write_copy(NCH - 1, (NCH - 1) % 2).wait()

    out = run(xf, emb, posenc)
    return out.reshape(B, T, D)
